# baseline (speedup 1.0000x reference)
import functools

import jax
import jax.numpy as jnp
from jax import lax
from jax.experimental import pallas as pl
from jax.experimental.pallas import tpu as pltpu
from jax.experimental.pallas import tpu_sc as plsc

N = 10000
E = 320000
D = 128
NC = 2
NS = 16
NW = NC * NS
EPT = E // NW
BT = 56
NB = 184
EPT_PAD = NB * BT
PAD_E = EPT_PAD - EPT
NPAD = N + 16
NACC = 2
RZ = NACC * NPAD // NS
RW = N // NS
NI = NB // 4


@functools.cache
def _make_sc_degrees():
    return functools.partial(
        pl.kernel,
        mesh=plsc.VectorSubcoreMesh(core_axis_name="c", subcore_axis_name="s"),
        out_type=[
            jax.ShapeDtypeStruct((NW, NPAD), jnp.float32),
            jax.ShapeDtypeStruct((NW, NPAD), jnp.float32),
        ],
        scratch_types=[
            pltpu.VMEM((EPT,), jnp.int32),
            pltpu.VMEM((EPT,), jnp.int32),
            pltpu.VMEM((NPAD,), jnp.float32),
            pltpu.VMEM((NPAD,), jnp.float32),
        ],
        compiler_params=pltpu.CompilerParams(needs_layout_passes=False),
    )(_sc_degrees_body)


def _sc_degrees_body(src_hbm, dst_hbm, dout_hbm, din_hbm, src_v, dst_v, do_v, di_v):
    c = lax.axis_index("c")
    s = lax.axis_index("s")
    w = c * NS + s
    pltpu.sync_copy(src_hbm.at[pl.ds(w * EPT, EPT)], src_v)
    pltpu.sync_copy(dst_hbm.at[pl.ds(w * EPT, EPT)], dst_v)

    zeros = jnp.zeros((16,), jnp.float32)

    def zbody(i, carry):
        do_v[pl.ds(i * 16, 16)] = zeros
        di_v[pl.ds(i * 16, 16)] = zeros
        return carry

    lax.fori_loop(0, NPAD // 16, zbody, 0)

    ones = jnp.ones((16,), jnp.float32)

    def body(i, carry):
        si = src_v[pl.ds(i * 16, 16)]
        di = dst_v[pl.ds(i * 16, 16)]
        plsc.addupdate_scatter(do_v, [si], ones)
        plsc.addupdate_scatter(di_v, [di], ones)
        return carry

    lax.fori_loop(0, EPT // 16, body, 0)

    pltpu.sync_copy(do_v, dout_hbm.at[w])
    pltpu.sync_copy(di_v, din_hbm.at[w])


@functools.cache
def _make_sc_aggregate():
    return functools.partial(
        pl.kernel,
        mesh=plsc.VectorSubcoreMesh(core_axis_name="c", subcore_axis_name="s"),
        out_type=jax.ShapeDtypeStruct((NC, NACC, N, D), jnp.bfloat16),
        scratch_types=[
            pltpu.VMEM((NB, BT), jnp.int32),
            pltpu.VMEM((NB, BT), jnp.int32),
            pltpu.VMEM((BT, D), jnp.bfloat16),
            pltpu.VMEM((BT, D), jnp.bfloat16),
            pltpu.VMEM((BT, D), jnp.bfloat16),
            pltpu.VMEM((BT, D), jnp.bfloat16),
            pltpu.VMEM((BT, D), jnp.bfloat16),
            pltpu.VMEM((BT, D), jnp.bfloat16),
            pltpu.VMEM((BT, D), jnp.bfloat16),
            pltpu.VMEM((BT, D), jnp.bfloat16),
            pltpu.VMEM_SHARED((NACC * NPAD, D), jnp.bfloat16),
        ] + [pltpu.SemaphoreType.DMA] * 16,
        compiler_params=pltpu.CompilerParams(
            needs_layout_passes=False, use_tc_tiling_on_sc=False
        ),
    )(_sc_aggregate_body)


def _sc_aggregate_body(tbl_hbm, srcp_hbm, dstp_hbm, out_hbm,
                       srcp_v, dstp_v, r0, r1, r2, r3, r4, r5, r6, r7, acc_sh,
                       *sems):
    c = lax.axis_index("c")
    s = lax.axis_index("s")
    w = c * NS + s
    tbl = tbl_hbm
    pltpu.sync_copy(srcp_hbm.at[w], srcp_v)
    pltpu.sync_copy(dstp_hbm.at[w], dstp_v)

    rows = [r0, r1, r2, r3, r4, r5, r6, r7]
    gsem = list(sems[:8])
    ssem = list(sems[8:])

    zeros = jnp.zeros((32,), jnp.bfloat16)

    def zbody(i, carry):
        r0[i // (D // 32), pl.ds((i % (D // 32)) * 32, 32)] = zeros
        return carry

    lax.fori_loop(0, BT * (D // 32), zbody, 0)

    base = s * RZ
    nfull = RZ // BT
    rem = RZ - nfull * BT

    def zcopy(k, carry):
        pltpu.sync_copy(r0, acc_sh.at[pl.ds(base + k * BT, BT)])
        return carry

    lax.fori_loop(0, nfull, zcopy, 0)
    pltpu.sync_copy(r0.at[pl.ds(0, rem)], acc_sh.at[pl.ds(base + nfull * BT, rem)])
    plsc.subcore_barrier()

    def gather(j, p):
        pltpu.async_copy(tbl.at[srcp_v.at[j]], rows[p], gsem[p])

    def gwait(j, p):
        pltpu.make_async_copy(tbl.at[srcp_v.at[j]], rows[p], gsem[p]).wait()

    def scat(j, p):
        pltpu.async_copy(rows[p], acc_sh.at[dstp_v.at[j]], ssem[p], add=True)

    def swait(j, p):
        pltpu.make_async_copy(rows[p], acc_sh.at[dstp_v.at[j]], ssem[p]).wait()

    gather(0, 0)
    gather(1, 1)
    gather(2, 2)
    gather(3, 3)

    def stage(i, j, p, head):
        gwait(j, p)
        scat(j, p)
        p4 = (p + 4) % 8

        def drain_and_refill():
            swait(j - 4, p4)

            @pl.when(j + 4 < NB)
            def _():
                gather(j + 4, p4)

        if head:
            @pl.when(i > 0)
            def _():
                drain_and_refill()

            @pl.when(i == 0)
            def _():
                gather(j + 4, p4)
        else:
            drain_and_refill()

    def body(i, carry):
        j0 = 8 * i
        for p in range(8):
            stage(i, j0 + p, p, p < 4)
        return carry

    lax.fori_loop(0, NB // 8, body, 0)
    for j in range(NB - 4, NB):
        swait(j, j % 8)
    plsc.subcore_barrier()
    for k in range(NACC):
        pltpu.sync_copy(
            acc_sh.at[pl.ds(k * NPAD + s * RW, RW)],
            out_hbm.at[c, k, pl.ds(s * RW, RW)],
        )


def _tc_norm_body(dop_ref, dip_ref, ns_ref, nd_ref):
    dsum_o = jnp.sum(dop_ref[...], axis=0, keepdims=True)
    dsum_i = jnp.sum(dip_ref[...], axis=0, keepdims=True)
    ns_ref[...] = jnp.where(dsum_o > 0.0, lax.rsqrt(jnp.maximum(dsum_o, 1.0)), 0.0)
    nd_ref[...] = jnp.where(dsum_i > 0.0, lax.rsqrt(jnp.maximum(dsum_i, 1.0)), 0.0)


_tc_norm = pl.pallas_call(
    _tc_norm_body,
    out_shape=[
        jax.ShapeDtypeStruct((1, NPAD), jnp.float32),
        jax.ShapeDtypeStruct((1, NPAD), jnp.float32),
    ],
)


def _tc_scale_body(x_ref, ns_ref, xs_ref):
    xs_ref[...] = (x_ref[...] * ns_ref[...]).astype(jnp.bfloat16)


_tc_scale = pl.pallas_call(
    _tc_scale_body,
    out_shape=jax.ShapeDtypeStruct((N, D), jnp.bfloat16),
)


def _sum_planes(p_ref):
    agg = p_ref[0, 0].astype(jnp.float32)
    for c in range(NC):
        for k in range(NACC):
            if c == 0 and k == 0:
                continue
            agg += p_ref[c, k].astype(jnp.float32)
    return agg


def _tc_mid_body(p_ref, nd_ref, ns_ref, w1_ref, b1_ref, hs_ref):
    agg = _sum_planes(p_ref) * nd_ref[...]
    h = jnp.dot(agg, w1_ref[...], preferred_element_type=jnp.float32) + b1_ref[...]
    hs_ref[...] = (h * ns_ref[...]).astype(jnp.bfloat16)


_tc_mid = pl.pallas_call(
    _tc_mid_body,
    out_shape=jax.ShapeDtypeStruct((N, D), jnp.bfloat16),
)


def _tc_final_body(p_ref, nd_ref, wm_ref, bm_ref, ws_ref, bs_ref, noise_ref, z_ref):
    agg = _sum_planes(p_ref) * nd_ref[...]
    mean = jnp.dot(agg, wm_ref[...], preferred_element_type=jnp.float32) + bm_ref[...]
    logstd = jnp.dot(agg, ws_ref[...], preferred_element_type=jnp.float32) + bs_ref[...]
    z_ref[...] = noise_ref[...] * jnp.exp(logstd) + mean


_tc_final = pl.pallas_call(
    _tc_final_body,
    out_shape=jax.ShapeDtypeStruct((N, D), jnp.float32),
)


def kernel(x, edge_index, W1, b1, Wm, bm, Ws, bs):
    src = edge_index[0].astype(jnp.int32)
    dst = edge_index[1].astype(jnp.int32)

    srcp = jnp.pad(src.reshape(NW, EPT), ((0, 0), (0, PAD_E))).reshape(NW, NB, BT)
    dpad = jnp.tile(jnp.arange(16, dtype=jnp.int32) + N, PAD_E // 16)
    dstp = jnp.concatenate(
        [dst.reshape(NW, EPT), jnp.broadcast_to(dpad, (NW, PAD_E))], axis=1
    ).reshape(NW, NB, BT)
    plane = (jnp.arange(NB, dtype=jnp.int32) % NACC) * NPAD
    dstp = dstp + plane[None, :, None]

    degp_out, degp_in = _make_sc_degrees()(src, dst)
    ns_row, nd_row = _tc_norm(degp_out, degp_in)
    ns = ns_row.reshape(NPAD, 1)[:N]
    nd = nd_row.reshape(NPAD, 1)[:N]

    xs = _tc_scale(x, ns)
    sc_agg = _make_sc_aggregate()
    agg1 = sc_agg(xs, srcp, dstp)
    hs = _tc_mid(agg1, nd, ns, W1, b1.reshape(1, D))
    agg2 = sc_agg(hs, srcp, dstp)

    noise = jax.random.normal(jax.random.key(42), (N, D), dtype=jnp.float32)
    z = _tc_final(agg2, nd, Wm, bm.reshape(1, D), Ws, bs.reshape(1, D), noise)
    return z


# speedup vs baseline: 1.0027x; 1.0027x over previous
import functools

import jax
import jax.numpy as jnp
from jax import lax
from jax.experimental import pallas as pl
from jax.experimental.pallas import tpu as pltpu
from jax.experimental.pallas import tpu_sc as plsc

N = 10000
E = 320000
D = 128
NC = 2
NS = 16
NW = NC * NS
EPT = E // NW
BT = 56
NB = 184
EPT_PAD = NB * BT
PAD_E = EPT_PAD - EPT
NPAD = N + 16
NACC = 2
RZ = NACC * NPAD // NS
RW = N // NS
NI = NB // 4


@functools.cache
def _make_sc_degrees():
    return functools.partial(
        pl.kernel,
        mesh=plsc.VectorSubcoreMesh(core_axis_name="c", subcore_axis_name="s"),
        out_type=[
            jax.ShapeDtypeStruct((NW, NPAD), jnp.float32),
            jax.ShapeDtypeStruct((NW, NPAD), jnp.float32),
        ],
        scratch_types=[
            pltpu.VMEM((EPT,), jnp.int32),
            pltpu.VMEM((EPT,), jnp.int32),
            pltpu.VMEM((NPAD,), jnp.float32),
            pltpu.VMEM((NPAD,), jnp.float32),
        ],
        compiler_params=pltpu.CompilerParams(needs_layout_passes=False),
    )(_sc_degrees_body)


def _sc_degrees_body(src_hbm, dst_hbm, dout_hbm, din_hbm, src_v, dst_v, do_v, di_v):
    c = lax.axis_index("c")
    s = lax.axis_index("s")
    w = c * NS + s
    pltpu.sync_copy(src_hbm.at[pl.ds(w * EPT, EPT)], src_v)
    pltpu.sync_copy(dst_hbm.at[pl.ds(w * EPT, EPT)], dst_v)

    zeros = jnp.zeros((16,), jnp.float32)

    def zbody(i, carry):
        do_v[pl.ds(i * 16, 16)] = zeros
        di_v[pl.ds(i * 16, 16)] = zeros
        return carry

    lax.fori_loop(0, NPAD // 16, zbody, 0)

    ones = jnp.ones((16,), jnp.float32)

    def body(i, carry):
        si = src_v[pl.ds(i * 16, 16)]
        di = dst_v[pl.ds(i * 16, 16)]
        plsc.addupdate_scatter(do_v, [si], ones)
        plsc.addupdate_scatter(di_v, [di], ones)
        return carry

    lax.fori_loop(0, EPT // 16, body, 0)

    pltpu.sync_copy(do_v, dout_hbm.at[w])
    pltpu.sync_copy(di_v, din_hbm.at[w])


@functools.cache
def _make_sc_aggregate():
    return functools.partial(
        pl.kernel,
        mesh=plsc.VectorSubcoreMesh(core_axis_name="c", subcore_axis_name="s"),
        out_type=jax.ShapeDtypeStruct((NC, NACC, N, D), jnp.bfloat16),
        scratch_types=[
            pltpu.VMEM((NB, BT), jnp.int32),
            pltpu.VMEM((NB, BT), jnp.int32),
            pltpu.VMEM((BT, D), jnp.bfloat16),
            pltpu.VMEM((BT, D), jnp.bfloat16),
            pltpu.VMEM((BT, D), jnp.bfloat16),
            pltpu.VMEM((BT, D), jnp.bfloat16),
            pltpu.VMEM((BT, D), jnp.bfloat16),
            pltpu.VMEM((BT, D), jnp.bfloat16),
            pltpu.VMEM((BT, D), jnp.bfloat16),
            pltpu.VMEM((BT, D), jnp.bfloat16),
            pltpu.VMEM_SHARED((NACC * NPAD, D), jnp.bfloat16),
        ] + [pltpu.SemaphoreType.DMA] * 16,
        compiler_params=pltpu.CompilerParams(
            needs_layout_passes=False, use_tc_tiling_on_sc=False
        ),
    )(_sc_aggregate_body)


def _sc_aggregate_body(tbl_hbm, srcp_hbm, dstp_hbm, out_hbm,
                       srcp_v, dstp_v, r0, r1, r2, r3, r4, r5, r6, r7, acc_sh,
                       *sems):
    c = lax.axis_index("c")
    s = lax.axis_index("s")
    w = c * NS + s
    tbl = tbl_hbm
    pltpu.sync_copy(srcp_hbm.at[w], srcp_v)
    pltpu.sync_copy(dstp_hbm.at[w], dstp_v)

    rows = [r0, r1, r2, r3, r4, r5, r6, r7]
    gsem = list(sems[:8])
    ssem = list(sems[8:])

    zeros = jnp.zeros((32,), jnp.bfloat16)

    def zbody(i, carry):
        r0[i // (D // 32), pl.ds((i % (D // 32)) * 32, 32)] = zeros
        return carry

    lax.fori_loop(0, BT * (D // 32), zbody, 0)

    base = s * RZ
    nfull = RZ // BT
    rem = RZ - nfull * BT

    def zcopy(k, carry):
        pltpu.sync_copy(r0, acc_sh.at[pl.ds(base + k * BT, BT)])
        return carry

    lax.fori_loop(0, nfull, zcopy, 0)
    pltpu.sync_copy(r0.at[pl.ds(0, rem)], acc_sh.at[pl.ds(base + nfull * BT, rem)])
    plsc.subcore_barrier()

    def gather(j, p):
        pltpu.async_copy(tbl.at[srcp_v.at[j]], rows[p], gsem[p])

    def gwait(j, p):
        pltpu.make_async_copy(tbl.at[srcp_v.at[j]], rows[p], gsem[p]).wait()

    def scat(j, p):
        pltpu.async_copy(rows[p], acc_sh.at[dstp_v.at[j]], ssem[p], add=True)

    def swait(j, p):
        pltpu.make_async_copy(rows[p], acc_sh.at[dstp_v.at[j]], ssem[p]).wait()

    gather(0, 0)
    gather(1, 1)
    gather(2, 2)
    gather(3, 3)

    def stage(i, j, p, head):
        gwait(j, p)
        scat(j, p)
        p4 = (p + 4) % 8

        def drain_and_refill():
            swait(j - 4, p4)

            @pl.when(j + 4 < NB)
            def _():
                gather(j + 4, p4)

        if head:
            @pl.when(i > 0)
            def _():
                drain_and_refill()

            @pl.when(i == 0)
            def _():
                gather(j + 4, p4)
        else:
            drain_and_refill()

    def body(i, carry):
        j0 = 8 * i
        for p in range(8):
            stage(i, j0 + p, p, p < 4)
        return carry

    lax.fori_loop(0, NB // 8, body, 0)
    for j in range(NB - 4, NB):
        swait(j, j % 8)
    plsc.subcore_barrier()
    for k in range(NACC):
        pltpu.sync_copy(
            acc_sh.at[pl.ds(k * NPAD + s * RW, RW)],
            out_hbm.at[c, k, pl.ds(s * RW, RW)],
        )


def _tc_norm_body(dop_ref, dip_ref, ns_ref, nd_ref):
    dsum_o = jnp.sum(dop_ref[...], axis=0, keepdims=True)
    dsum_i = jnp.sum(dip_ref[...], axis=0, keepdims=True)
    ns_ref[...] = jnp.where(dsum_o > 0.0, lax.rsqrt(jnp.maximum(dsum_o, 1.0)), 0.0)
    nd_ref[...] = jnp.where(dsum_i > 0.0, lax.rsqrt(jnp.maximum(dsum_i, 1.0)), 0.0)


_tc_norm = pl.pallas_call(
    _tc_norm_body,
    out_shape=[
        jax.ShapeDtypeStruct((1, NPAD), jnp.float32),
        jax.ShapeDtypeStruct((1, NPAD), jnp.float32),
    ],
)


def _tc_scale_body(x_ref, ns_ref, xs_ref):
    xs_ref[...] = (x_ref[...] * ns_ref[...]).astype(jnp.bfloat16)


_tc_scale = pl.pallas_call(
    _tc_scale_body,
    out_shape=jax.ShapeDtypeStruct((N, D), jnp.bfloat16),
)


def _sum_planes(p_ref):
    agg = p_ref[0, 0].astype(jnp.float32)
    for c in range(NC):
        for k in range(NACC):
            if c == 0 and k == 0:
                continue
            agg += p_ref[c, k].astype(jnp.float32)
    return agg


def _tc_mid_body(p_ref, nd_ref, ns_ref, w1_ref, b1_ref, hs_ref):
    agg = _sum_planes(p_ref) * nd_ref[...]
    h = jnp.dot(agg, w1_ref[...], preferred_element_type=jnp.float32) + b1_ref[...]
    hs_ref[...] = (h * ns_ref[...]).astype(jnp.bfloat16)


_tc_mid = pl.pallas_call(
    _tc_mid_body,
    out_shape=jax.ShapeDtypeStruct((N, D), jnp.bfloat16),
)


def _tc_final_body(p_ref, nd_ref, wm_ref, bm_ref, ws_ref, bs_ref, noise_ref, z_ref):
    agg = _sum_planes(p_ref) * nd_ref[...]
    mean = jnp.dot(agg, wm_ref[...], preferred_element_type=jnp.float32) + bm_ref[...]
    logstd = jnp.dot(agg, ws_ref[...], preferred_element_type=jnp.float32) + bs_ref[...]
    z_ref[...] = noise_ref[...] * jnp.exp(logstd) + mean


_tc_final = pl.pallas_call(
    _tc_final_body,
    out_shape=jax.ShapeDtypeStruct((N, D), jnp.float32),
)


def kernel(x, edge_index, W1, b1, Wm, bm, Ws, bs):
    src = edge_index[0].astype(jnp.int32)
    dst = edge_index[1].astype(jnp.int32)

    srcp = jnp.pad(src.reshape(NW, EPT), ((0, 0), (0, PAD_E))).reshape(NW, NB, BT)
    dpad = (jnp.arange(NW, dtype=jnp.int32) % 16 + N)[:, None]
    dstp = jnp.concatenate(
        [dst.reshape(NW, EPT), jnp.broadcast_to(dpad, (NW, PAD_E))], axis=1
    ).reshape(NW, NB, BT)
    plane = (jnp.arange(NB, dtype=jnp.int32) % NACC) * NPAD
    dstp = dstp + plane[None, :, None]

    degp_out, degp_in = _make_sc_degrees()(src, dst)
    ns_row, nd_row = _tc_norm(degp_out, degp_in)
    ns = ns_row.reshape(NPAD, 1)[:N]
    nd = nd_row.reshape(NPAD, 1)[:N]

    xs = _tc_scale(x, ns)
    sc_agg = _make_sc_aggregate()
    agg1 = sc_agg(xs, srcp, dstp)
    hs = _tc_mid(agg1, nd, ns, W1, b1.reshape(1, D))
    agg2 = sc_agg(hs, srcp, dstp)

    noise = jax.random.normal(jax.random.key(42), (N, D), dtype=jnp.float32)
    z = _tc_final(agg2, nd, Wm, bm.reshape(1, D), Ws, bs.reshape(1, D), noise)
    return z


# speedup vs baseline: 1.5977x; 1.5934x over previous
import functools

import jax
import jax.numpy as jnp
from jax import lax
from jax.experimental import pallas as pl
from jax.experimental.pallas import tpu as pltpu
from jax.experimental.pallas import tpu_sc as plsc

N = 10000
E = 320000
D = 128
NC = 2
NS = 16
NW = NC * NS
EPT = E // NW
BT = 56
NB = 180
EPT_PAD = NB * BT
PAD_E = EPT_PAD - EPT
NPAD = N + 16
NACC = 2
RZ = NACC * NPAD // NS
RW = N // NS
NI = NB // 4


@functools.cache
def _make_sc_degrees():
    return functools.partial(
        pl.kernel,
        mesh=plsc.VectorSubcoreMesh(core_axis_name="c", subcore_axis_name="s"),
        out_type=[
            jax.ShapeDtypeStruct((NW, NPAD), jnp.float32),
            jax.ShapeDtypeStruct((NW, NPAD), jnp.float32),
        ],
        scratch_types=[
            pltpu.VMEM((EPT,), jnp.int32),
            pltpu.VMEM((EPT,), jnp.int32),
            pltpu.VMEM((NPAD,), jnp.float32),
            pltpu.VMEM((NPAD,), jnp.float32),
        ],
        compiler_params=pltpu.CompilerParams(needs_layout_passes=False),
    )(_sc_degrees_body)


def _sc_degrees_body(src_hbm, dst_hbm, dout_hbm, din_hbm, src_v, dst_v, do_v, di_v):
    c = lax.axis_index("c")
    s = lax.axis_index("s")
    w = c * NS + s
    pltpu.sync_copy(src_hbm.at[pl.ds(w * EPT, EPT)], src_v)
    pltpu.sync_copy(dst_hbm.at[pl.ds(w * EPT, EPT)], dst_v)

    zeros = jnp.zeros((16,), jnp.float32)

    def zbody(i, carry):
        do_v[pl.ds(i * 16, 16)] = zeros
        di_v[pl.ds(i * 16, 16)] = zeros
        return carry

    lax.fori_loop(0, NPAD // 16, zbody, 0)

    ones = jnp.ones((16,), jnp.float32)

    def body(i, carry):
        si = src_v[pl.ds(i * 16, 16)]
        di = dst_v[pl.ds(i * 16, 16)]
        plsc.addupdate_scatter(do_v, [si], ones)
        plsc.addupdate_scatter(di_v, [di], ones)
        return carry

    lax.fori_loop(0, EPT // 16, body, 0)

    pltpu.sync_copy(do_v, dout_hbm.at[w])
    pltpu.sync_copy(di_v, din_hbm.at[w])


@functools.cache
def _make_sc_aggregate():
    return functools.partial(
        pl.kernel,
        mesh=plsc.VectorSubcoreMesh(core_axis_name="c", subcore_axis_name="s"),
        out_type=jax.ShapeDtypeStruct((NC, NACC, N, D), jnp.bfloat16),
        scratch_types=[
            pltpu.VMEM((NB, BT), jnp.int32),
            pltpu.VMEM((NB, BT), jnp.int32),
            pltpu.VMEM((BT, D), jnp.bfloat16),
            pltpu.VMEM((BT, D), jnp.bfloat16),
            pltpu.VMEM((BT, D), jnp.bfloat16),
            pltpu.VMEM((BT, D), jnp.bfloat16),
            pltpu.VMEM((BT, D), jnp.bfloat16),
            pltpu.VMEM((BT, D), jnp.bfloat16),
            pltpu.VMEM_SHARED((NACC * NPAD, D), jnp.bfloat16),
        ] + [pltpu.SemaphoreType.DMA] * 12,
        compiler_params=pltpu.CompilerParams(
            needs_layout_passes=False, use_tc_tiling_on_sc=False
        ),
    )(_sc_aggregate_body)


def _sc_aggregate_body(tbl_hbm, srcp_hbm, dstp_hbm, out_hbm,
                       srcp_v, dstp_v, r0, r1, r2, r3, r4, r5, acc_sh,
                       *sems):
    c = lax.axis_index("c")
    s = lax.axis_index("s")
    w = c * NS + s
    tbl = tbl_hbm
    pltpu.sync_copy(srcp_hbm.at[w], srcp_v)
    pltpu.sync_copy(dstp_hbm.at[w], dstp_v)

    rows = [r0, r1, r2, r3, r4, r5]
    gsem = list(sems[:6])
    ssem = list(sems[6:])

    zeros = jnp.zeros((32,), jnp.bfloat16)

    def zbody(i, carry):
        r0[i // (D // 32), pl.ds((i % (D // 32)) * 32, 32)] = zeros
        return carry

    lax.fori_loop(0, BT * (D // 32), zbody, 0)

    base = s * RZ
    nfull = RZ // BT
    rem = RZ - nfull * BT

    def zcopy(k, carry):
        pltpu.sync_copy(r0, acc_sh.at[pl.ds(base + k * BT, BT)])
        return carry

    lax.fori_loop(0, nfull, zcopy, 0)
    pltpu.sync_copy(r0.at[pl.ds(0, rem)], acc_sh.at[pl.ds(base + nfull * BT, rem)])
    plsc.subcore_barrier()

    def gather(j, p):
        pltpu.async_copy(tbl.at[srcp_v.at[j]], rows[p], gsem[p])

    def gwait(j, p):
        pltpu.make_async_copy(tbl.at[srcp_v.at[j]], rows[p], gsem[p]).wait()

    def scat(j, p):
        pltpu.async_copy(rows[p], acc_sh.at[dstp_v.at[j]], ssem[p], add=True)

    def swait(j, p):
        pltpu.make_async_copy(rows[p], acc_sh.at[dstp_v.at[j]], ssem[p]).wait()

    gather(0, 0)
    gather(1, 1)
    gather(2, 2)

    def stage(i, j, p, head):
        gwait(j, p)
        scat(j, p)
        p3 = (p + 3) % 6

        def drain_and_refill():
            swait(j - 3, p3)

            @pl.when(j + 3 < NB)
            def _():
                gather(j + 3, p3)

        if head:
            @pl.when(i > 0)
            def _():
                drain_and_refill()

            @pl.when(i == 0)
            def _():
                gather(j + 3, p3)
        else:
            drain_and_refill()

    def body(i, carry):
        j0 = 6 * i
        for p in range(6):
            stage(i, j0 + p, p, p < 3)
        return carry

    lax.fori_loop(0, NB // 6, body, 0)
    for j in range(NB - 3, NB):
        swait(j, j % 6)
    plsc.subcore_barrier()
    for k in range(NACC):
        pltpu.sync_copy(
            acc_sh.at[pl.ds(k * NPAD + s * RW, RW)],
            out_hbm.at[c, k, pl.ds(s * RW, RW)],
        )


def _tc_norm_body(dop_ref, dip_ref, ns_ref, nd_ref):
    dsum_o = jnp.sum(dop_ref[...], axis=0, keepdims=True)
    dsum_i = jnp.sum(dip_ref[...], axis=0, keepdims=True)
    ns_ref[...] = jnp.where(dsum_o > 0.0, lax.rsqrt(jnp.maximum(dsum_o, 1.0)), 0.0)
    nd_ref[...] = jnp.where(dsum_i > 0.0, lax.rsqrt(jnp.maximum(dsum_i, 1.0)), 0.0)


_tc_norm = pl.pallas_call(
    _tc_norm_body,
    out_shape=[
        jax.ShapeDtypeStruct((1, NPAD), jnp.float32),
        jax.ShapeDtypeStruct((1, NPAD), jnp.float32),
    ],
)


def _tc_scale_body(x_ref, ns_ref, xs_ref):
    xs_ref[...] = (x_ref[...] * ns_ref[...]).astype(jnp.bfloat16)


_tc_scale = pl.pallas_call(
    _tc_scale_body,
    out_shape=jax.ShapeDtypeStruct((N, D), jnp.bfloat16),
)


def _sum_planes(p_ref):
    agg = p_ref[0, 0].astype(jnp.float32)
    for c in range(NC):
        for k in range(NACC):
            if c == 0 and k == 0:
                continue
            agg += p_ref[c, k].astype(jnp.float32)
    return agg


def _tc_mid_body(p_ref, nd_ref, ns_ref, w1_ref, b1_ref, hs_ref):
    agg = _sum_planes(p_ref) * nd_ref[...]
    h = jnp.dot(agg, w1_ref[...], preferred_element_type=jnp.float32) + b1_ref[...]
    hs_ref[...] = (h * ns_ref[...]).astype(jnp.bfloat16)


_tc_mid = pl.pallas_call(
    _tc_mid_body,
    out_shape=jax.ShapeDtypeStruct((N, D), jnp.bfloat16),
)


def _tc_final_body(p_ref, nd_ref, wm_ref, bm_ref, ws_ref, bs_ref, noise_ref, z_ref):
    agg = _sum_planes(p_ref) * nd_ref[...]
    mean = jnp.dot(agg, wm_ref[...], preferred_element_type=jnp.float32) + bm_ref[...]
    logstd = jnp.dot(agg, ws_ref[...], preferred_element_type=jnp.float32) + bs_ref[...]
    z_ref[...] = noise_ref[...] * jnp.exp(logstd) + mean


_tc_final = pl.pallas_call(
    _tc_final_body,
    out_shape=jax.ShapeDtypeStruct((N, D), jnp.float32),
)


def kernel(x, edge_index, W1, b1, Wm, bm, Ws, bs):
    src = edge_index[0].astype(jnp.int32)
    dst = edge_index[1].astype(jnp.int32)

    srcp = jnp.pad(src.reshape(NW, EPT), ((0, 0), (0, PAD_E))).reshape(NW, NB, BT)
    dpad = (jnp.arange(NW, dtype=jnp.int32) % 16 + N)[:, None]
    dstp = jnp.concatenate(
        [dst.reshape(NW, EPT), jnp.broadcast_to(dpad, (NW, PAD_E))], axis=1
    ).reshape(NW, NB, BT)
    plane = (jnp.arange(NB, dtype=jnp.int32) % NACC) * NPAD
    dstp = dstp + plane[None, :, None]

    degp_out, degp_in = _make_sc_degrees()(src, dst)
    ns_row, nd_row = _tc_norm(degp_out, degp_in)
    ns = ns_row.reshape(NPAD, 1)[:N]
    nd = nd_row.reshape(NPAD, 1)[:N]

    xs = _tc_scale(x, ns)
    sc_agg = _make_sc_aggregate()
    agg1 = sc_agg(xs, srcp, dstp)
    hs = _tc_mid(agg1, nd, ns, W1, b1.reshape(1, D))
    agg2 = sc_agg(hs, srcp, dstp)

    noise = jax.random.normal(jax.random.key(42), (N, D), dtype=jnp.float32)
    z = _tc_final(agg2, nd, Wm, bm.reshape(1, D), Ws, bs.reshape(1, D), noise)
    return z
